# Initial kernel scaffold; baseline (speedup 1.0000x reference)
import jax
import jax.numpy as jnp
from jax.experimental import pallas as pl


def kernel(input, label):
    raise NotImplementedError("write your pallas kernel here")


# speedup vs baseline: 1.8251x; 1.8251x over previous
import functools
import math

import jax
import jax.numpy as jnp
from jax.experimental import pallas as pl
from jax.experimental.pallas import tpu as pltpu

_M = 4
_COS_M = math.cos(_M)
_SIN_M = math.sin(_M)


def _body(x_ref, lbl_ref, out_ref, acc_ref, gacc_ref, *, c):
    j = pl.program_id(0)
    nj = pl.num_programs(0)
    bc = x_ref.shape[1]

    @pl.when(j == 0)
    def _init():
        acc_ref[0, 0] = 0.0
        gacc_ref[...] = jnp.zeros_like(gacc_ref)

    def accumulate(x):
        acc_ref[0, 0] += jnp.sum(x * x)
        rel = lbl_ref[...] - j * bc
        col = jax.lax.broadcasted_iota(jnp.int32, x.shape, 1)
        g = jnp.sum(jnp.where(col == rel, x, 0.0), axis=1, keepdims=True)
        gacc_ref[...] += g

    @pl.when(j < nj - 1)
    def _interior():
        accumulate(x_ref[...])

    @pl.when(j == nj - 1)
    def _last():
        x = x_ref[...]
        col = jax.lax.broadcasted_iota(jnp.int32, x.shape, 1)
        accumulate(jnp.where(col < c - j * bc, x, 0.0))
        v = gacc_ref[...]
        phi = v * _COS_M - jnp.sqrt(jnp.maximum(1.0 - v * v, 0.0)) * _SIN_M
        corr = jnp.sum(phi * phi - v * v)
        total_n = gacc_ref.shape[0] * c
        out_ref[0, 0] = (acc_ref[0, 0] + corr) * (_M * _M / total_n)


def kernel(input, label):
    b, c = input.shape
    bc = 2560
    grid = (pl.cdiv(c, bc),)
    lbl = label.astype(jnp.int32).reshape(b, 1)

    out = pl.pallas_call(
        functools.partial(_body, c=c),
        grid=grid,
        in_specs=[
            pl.BlockSpec((b, bc), lambda j: (0, j)),
            pl.BlockSpec((b, 1), lambda j: (0, 0)),
        ],
        out_specs=pl.BlockSpec(memory_space=pltpu.SMEM),
        out_shape=jax.ShapeDtypeStruct((1, 1), jnp.float32),
        scratch_shapes=[
            pltpu.SMEM((1, 1), jnp.float32),
            pltpu.VMEM((b, 1), jnp.float32),
        ],
    )(input, lbl)
    return out.reshape(())
